# baseline (speedup 1.0000x reference)
import math

import jax
import jax.numpy as jnp
from jax import lax
from jax.experimental import pallas as pl
from jax.experimental.pallas import tpu as pltpu
from jax.experimental.pallas import tpu_sc as plsc

N = 10000
K = 32
D = 128
B = 16384

NC = 2
NS = 16
NW = NC * NS
L = 16

NPW = 320
NPAD = NPW * NW
CH = 2
CHK = CH * K
NCH = NPW // CH
NBUF = 4

PPW = B // NW
PCH = 128
NPCH = PPW // PCH

SCALE = math.exp(-1.0)
NSUB = D // L


def _tc_prep_body(w_ref, o_ref):
    w = w_ref[...]
    s = jnp.sum(w, axis=1, keepdims=True)
    o_ref[...] = w * (SCALE / (s + 1e-9))


def _tc_prep(edge_w_p):
    return pl.pallas_call(
        _tc_prep_body,
        out_shape=jax.ShapeDtypeStruct((NPAD, K), jnp.float32),
    )(edge_w_p)


def _sc_agg_body(h_hbm, nbf_hbm, wf_hbm, out_hbm,
                 idx_v, w_v, rows0, rows1, rows2, rows3, out_v,
                 sem0, sem1, sem2, sem3):
    wid = lax.axis_index("s") * NC + lax.axis_index("c")
    base = wid * NPW

    pltpu.sync_copy(nbf_hbm.at[pl.ds(base * K, NPW * K)], idx_v)
    pltpu.sync_copy(wf_hbm.at[pl.ds(base * K, NPW * K)], w_v)

    rows = (rows0, rows1, rows2, rows3)
    sems = (sem0, sem1, sem2, sem3)

    def fire(c, bufi):
        cc = jnp.minimum(c, NCH - 1)
        pltpu.async_copy(h_hbm.at[idx_v.at[pl.ds(cc * CHK, CHK)]],
                         rows[bufi], sems[bufi])

    def wait(bufi):
        pltpu.make_async_copy(h_hbm.at[idx_v.at[pl.ds(0, CHK)]],
                              rows[bufi], sems[bufi]).wait()

    def compute(c, bufi):
        r_ref = rows[bufi]
        for bnode in range(CH):
            node = c * CH + bnode
            p = node * K
            wv = [w_v[pl.ds(p + j * L, L)] for j in range(K // L)]
            accs = [jnp.zeros((L,), jnp.float32) for _ in range(NSUB)]
            for k in range(K):
                wk = wv[k // L][k % L]
                row = bnode * K + k
                for sub in range(NSUB):
                    accs[sub] = accs[sub] + wk * r_ref[row, pl.ds(sub * L, L)]
            for sub in range(NSUB):
                out_v[node, pl.ds(sub * L, L)] = accs[sub]

    for b in range(NBUF - 1):
        fire(b, b)

    def body(i, carry):
        c0 = i * NBUF
        for b in range(NBUF):
            wait(b)
            compute(c0 + b, b)
            fire(c0 + b + NBUF - 1, (b + NBUF - 1) % NBUF)
        return carry

    lax.fori_loop(0, NCH // NBUF - 1, body, 0)
    c0 = NCH - NBUF
    for b in range(NBUF):
        wait(b)
        compute(c0 + b, b)
        if c0 + b + NBUF - 1 < NCH:
            fire(c0 + b + NBUF - 1, (b + NBUF - 1) % NBUF)
    pltpu.sync_copy(out_v, out_hbm.at[pl.ds(base, NPW)])


_sc_agg = pl.kernel(
    _sc_agg_body,
    out_type=jax.ShapeDtypeStruct((NPAD, D), jnp.float32),
    mesh=plsc.VectorSubcoreMesh(core_axis_name="c", subcore_axis_name="s",
                                num_cores=NC, num_subcores=NS),
    scratch_types=[
        pltpu.VMEM((NPW * K,), jnp.int32),
        pltpu.VMEM((NPW * K,), jnp.float32),
        pltpu.VMEM((CHK, D), jnp.float32),
        pltpu.VMEM((CHK, D), jnp.float32),
        pltpu.VMEM((CHK, D), jnp.float32),
        pltpu.VMEM((CHK, D), jnp.float32),
        pltpu.VMEM((NPW, D), jnp.float32),
        pltpu.SemaphoreType.DMA,
        pltpu.SemaphoreType.DMA,
        pltpu.SemaphoreType.DMA,
        pltpu.SemaphoreType.DMA,
    ],
)


def _sc_pair_body(a_hbm, b_hbm, src_hbm, dst_hbm, ao_hbm, bo_hbm,
                  sidx_v, didx_v, bufa, bufb, sem_a, sem_b):
    wid = lax.axis_index("s") * NC + lax.axis_index("c")
    base = wid * PPW
    pltpu.sync_copy(src_hbm.at[pl.ds(base, PPW)], sidx_v)
    pltpu.sync_copy(dst_hbm.at[pl.ds(base, PPW)], didx_v)

    for c in range(NPCH):
        pltpu.async_copy(a_hbm.at[sidx_v.at[pl.ds(c * PCH, PCH)]],
                         bufa, sem_a)
        pltpu.async_copy(b_hbm.at[didx_v.at[pl.ds(c * PCH, PCH)]],
                         bufb, sem_b)
        pltpu.make_async_copy(a_hbm.at[sidx_v.at[pl.ds(c * PCH, PCH)]],
                              bufa, sem_a).wait()
        pltpu.sync_copy(bufa, ao_hbm.at[pl.ds(base + c * PCH, PCH)])
        pltpu.make_async_copy(b_hbm.at[didx_v.at[pl.ds(c * PCH, PCH)]],
                              bufb, sem_b).wait()
        pltpu.sync_copy(bufb, bo_hbm.at[pl.ds(base + c * PCH, PCH)])


_sc_pair = pl.kernel(
    _sc_pair_body,
    out_type=[
        jax.ShapeDtypeStruct((B, D), jnp.float32),
        jax.ShapeDtypeStruct((B, D), jnp.float32),
    ],
    mesh=plsc.VectorSubcoreMesh(core_axis_name="c", subcore_axis_name="s",
                                num_cores=NC, num_subcores=NS),
    scratch_types=[
        pltpu.VMEM((PPW,), jnp.int32),
        pltpu.VMEM((PPW,), jnp.int32),
        pltpu.VMEM((PCH, D), jnp.float32),
        pltpu.VMEM((PCH, D), jnp.float32),
        pltpu.SemaphoreType.DMA,
        pltpu.SemaphoreType.DMA,
    ],
)


TCR = 1024


def _tc_layer1_body(x_ref, g_ref, w_ref, o_ref):
    o_ref[...] = jnp.tanh(
        jnp.dot(x_ref[...] + g_ref[...], w_ref[...],
                preferred_element_type=jnp.float32))


def _tc_layer1(x, agg, w):
    return pl.pallas_call(
        _tc_layer1_body,
        grid=(NPAD // TCR,),
        in_specs=[
            pl.BlockSpec((TCR, D), lambda i: (i, 0)),
            pl.BlockSpec((TCR, D), lambda i: (i, 0)),
            pl.BlockSpec((D, D), lambda i: (0, 0)),
        ],
        out_specs=pl.BlockSpec((TCR, D), lambda i: (i, 0)),
        out_shape=jax.ShapeDtypeStruct((NPAD, D), jnp.float32),
    )(x, agg, w)


def _tc_layer2_body(x_ref, g_ref, w_ref, wh_ref, a_ref, b_ref):
    h2 = jnp.tanh(
        jnp.dot(x_ref[...] + g_ref[...], w_ref[...],
                preferred_element_type=jnp.float32))
    a_ref[...] = jnp.dot(h2, wh_ref[0:D, :],
                         preferred_element_type=jnp.float32)
    b_ref[...] = jnp.dot(h2, wh_ref[D:2 * D, :],
                         preferred_element_type=jnp.float32)


def _tc_layer2(x, agg, w, wh):
    return pl.pallas_call(
        _tc_layer2_body,
        grid=(NPAD // TCR,),
        in_specs=[
            pl.BlockSpec((TCR, D), lambda i: (i, 0)),
            pl.BlockSpec((TCR, D), lambda i: (i, 0)),
            pl.BlockSpec((D, D), lambda i: (0, 0)),
            pl.BlockSpec((2 * D, D), lambda i: (0, 0)),
        ],
        out_specs=[
            pl.BlockSpec((TCR, D), lambda i: (i, 0)),
            pl.BlockSpec((TCR, D), lambda i: (i, 0)),
        ],
        out_shape=[
            jax.ShapeDtypeStruct((NPAD, D), jnp.float32),
            jax.ShapeDtypeStruct((NPAD, D), jnp.float32),
        ],
    )(x, agg, w, wh)


BR = 2048


def _tc_head_body(a_ref, b_ref, lab_ref, bh_ref, wot_ref, bo_ref, loss_ref):
    h = jnp.maximum(a_ref[...] + b_ref[...] + bh_ref[...], 0.0)
    l0 = jnp.sum(h * wot_ref[0:1, :], axis=1, keepdims=True) + bo_ref[0]
    l1 = jnp.sum(h * wot_ref[1:2, :], axis=1, keepdims=True) + bo_ref[1]
    m = jnp.maximum(l0, l1)
    lse = m + jnp.log(jnp.exp(l0 - m) + jnp.exp(l1 - m))
    sel = jnp.where(lab_ref[...] == 0, l0, l1)
    part = jnp.sum(lse - sel)

    @pl.when(pl.program_id(0) == 0)
    def _():
        loss_ref[0, 0] = 0.0

    loss_ref[0, 0] += part * (1.0 / B)


def _tc_head(a, b, labels2d, bh2d, wot, bo):
    return pl.pallas_call(
        _tc_head_body,
        grid=(B // BR,),
        in_specs=[
            pl.BlockSpec((BR, D), lambda i: (i, 0)),
            pl.BlockSpec((BR, D), lambda i: (i, 0)),
            pl.BlockSpec((BR, 1), lambda i: (i, 0)),
            pl.BlockSpec((1, D), lambda i: (0, 0)),
            pl.BlockSpec((2, D), lambda i: (0, 0)),
            pl.BlockSpec(memory_space=pltpu.SMEM),
        ],
        out_specs=pl.BlockSpec(memory_space=pltpu.SMEM),
        out_shape=jax.ShapeDtypeStruct((1, 1), jnp.float32),
    )(a, b, labels2d, bh2d, wot, bo)


def kernel(pairs, labels, neighbors, edge_w, emb, W0, W1, Wh, bh, Wo, bo):
    emb_p = jnp.pad(emb, ((0, NPAD - N), (0, 0)))
    nbf = jnp.pad(neighbors, ((0, NPAD - N), (0, 0))).reshape(-1)
    ew_p = jnp.pad(edge_w, ((0, NPAD - N), (0, 0)))

    wf = _tc_prep(ew_p).reshape(-1)

    agg1 = _sc_agg(emb_p, nbf, wf)
    h1 = _tc_layer1(emb_p, agg1, W0)
    agg2 = _sc_agg(h1, nbf, wf)
    a, b = _tc_layer2(h1, agg2, W1, Wh)

    ar, br = _sc_pair(a, b, pairs[:, 0], pairs[:, 1])

    loss = _tc_head(ar, br, labels.reshape(B, 1), bh.reshape(1, D),
                    Wo.T, bo)
    return loss[0, 0]


# speedup vs baseline: 3.5463x; 3.5463x over previous
import math

import jax
import jax.numpy as jnp
from jax import lax
from jax.experimental import pallas as pl
from jax.experimental.pallas import tpu as pltpu
from jax.experimental.pallas import tpu_sc as plsc

N = 10000
K = 32
D = 128
B = 16384

NC = 2
NS = 16
NW = NC * NS
L = 16

NPW = 320
NPAD = NPW * NW
CH = 2
CHK = CH * K
NCH = NPW // CH
NBUF = 2

PPW = B // NW
PCH = 128
NPCH = PPW // PCH

SCALE = math.exp(-1.0)
NSUB = D // L


def _tc_prep_body(w_ref, o_ref):
    w = w_ref[...]
    s = jnp.sum(w, axis=1, keepdims=True)
    o_ref[...] = w * (SCALE / (s + 1e-9))


def _tc_prep(edge_w_p):
    return pl.pallas_call(
        _tc_prep_body,
        out_shape=jax.ShapeDtypeStruct((NPAD, K), jnp.float32),
    )(edge_w_p)


def _sc_agg_body(h_hbm, nbf_hbm, wf_hbm, out_hbm,
                 h_sp, idx_v, w_v, rows0, rows1, ob0, ob1,
                 sem0, sem1, semo0, semo1):
    sid = lax.axis_index("s")
    wid = sid * NC + lax.axis_index("c")
    base = wid * NPW

    @pl.when(sid == 0)
    def _():
        pltpu.sync_copy(h_hbm, h_sp)

    pltpu.sync_copy(nbf_hbm.at[pl.ds(base * K, NPW * K)], idx_v)
    pltpu.sync_copy(wf_hbm.at[pl.ds(base * K, NPW * K)], w_v)
    plsc.subcore_barrier()

    rows = (rows0, rows1)
    sems = (sem0, sem1)
    obs = (ob0, ob1)
    osems = (semo0, semo1)

    def fire(c, bufi):
        pltpu.async_copy(h_sp.at[idx_v.at[pl.ds(c * CHK, CHK)]],
                         rows[bufi], sems[bufi])

    def wait(bufi):
        pltpu.make_async_copy(h_hbm.at[idx_v.at[pl.ds(0, CHK)]],
                              rows[bufi], sems[bufi]).wait()

    def ofire(c, b):
        pltpu.async_copy(obs[b], out_hbm.at[pl.ds(base + c * CH, CH)],
                         osems[b])

    def owait(c, b):
        pltpu.make_async_copy(obs[b], out_hbm.at[pl.ds(base + c * CH, CH)],
                              osems[b]).wait()

    def compute(c, bufi):
        r_ref = rows[bufi]
        o_ref = obs[bufi]
        for bnode in range(CH):
            node = c * CH + bnode
            p = node * K
            wv = [w_v[pl.ds(p + j * L, L)] for j in range(K // L)]
            accs = [jnp.zeros((L,), jnp.float32) for _ in range(NSUB)]
            for k in range(K):
                wk = wv[k // L][k % L]
                row = bnode * K + k
                for sub in range(NSUB):
                    accs[sub] = accs[sub] + wk * r_ref[row, pl.ds(sub * L, L)]
            for sub in range(NSUB):
                o_ref[bnode, pl.ds(sub * L, L)] = accs[sub]

    for b in range(NBUF):
        fire(b, b)

    def body(i, carry):
        c0 = i * NBUF
        for b in range(NBUF):
            c = c0 + b
            wait(b)

            @pl.when(c >= NBUF)
            def _():
                owait(c - NBUF, b)

            compute(c, b)
            ofire(c, b)

            @pl.when(c + NBUF < NCH)
            def _():
                fire(c + NBUF, b)
        return carry

    lax.fori_loop(0, NCH // NBUF, body, 0)
    for b in range(NBUF):
        owait(NCH - NBUF + b, b)


_sc_agg = pl.kernel(
    _sc_agg_body,
    out_type=jax.ShapeDtypeStruct((NPAD, D), jnp.float32),
    mesh=plsc.VectorSubcoreMesh(core_axis_name="c", subcore_axis_name="s",
                                num_cores=NC, num_subcores=NS),
    scratch_types=[
        pltpu.VMEM_SHARED((NPAD, D), jnp.float32),
        pltpu.VMEM((NPW * K,), jnp.int32),
        pltpu.VMEM((NPW * K,), jnp.float32),
        pltpu.VMEM((CHK, D), jnp.float32),
        pltpu.VMEM((CHK, D), jnp.float32),
        pltpu.VMEM((CH, D), jnp.float32),
        pltpu.VMEM((CH, D), jnp.float32),
        pltpu.SemaphoreType.DMA,
        pltpu.SemaphoreType.DMA,
        pltpu.SemaphoreType.DMA,
        pltpu.SemaphoreType.DMA,
    ],
)


def _sc_pair_body(a_hbm, b_hbm, src_hbm, dst_hbm, ao_hbm, bo_hbm,
                  sidx_v, didx_v, bufa, bufb, sem_a, sem_b):
    wid = lax.axis_index("s") * NC + lax.axis_index("c")
    base = wid * PPW
    pltpu.sync_copy(src_hbm.at[pl.ds(base, PPW)], sidx_v)
    pltpu.sync_copy(dst_hbm.at[pl.ds(base, PPW)], didx_v)

    for c in range(NPCH):
        pltpu.async_copy(a_hbm.at[sidx_v.at[pl.ds(c * PCH, PCH)]],
                         bufa, sem_a)
        pltpu.async_copy(b_hbm.at[didx_v.at[pl.ds(c * PCH, PCH)]],
                         bufb, sem_b)
        pltpu.make_async_copy(a_hbm.at[sidx_v.at[pl.ds(c * PCH, PCH)]],
                              bufa, sem_a).wait()
        pltpu.sync_copy(bufa, ao_hbm.at[pl.ds(base + c * PCH, PCH)])
        pltpu.make_async_copy(b_hbm.at[didx_v.at[pl.ds(c * PCH, PCH)]],
                              bufb, sem_b).wait()
        pltpu.sync_copy(bufb, bo_hbm.at[pl.ds(base + c * PCH, PCH)])


_sc_pair = pl.kernel(
    _sc_pair_body,
    out_type=[
        jax.ShapeDtypeStruct((B, D), jnp.float32),
        jax.ShapeDtypeStruct((B, D), jnp.float32),
    ],
    mesh=plsc.VectorSubcoreMesh(core_axis_name="c", subcore_axis_name="s",
                                num_cores=NC, num_subcores=NS),
    scratch_types=[
        pltpu.VMEM((PPW,), jnp.int32),
        pltpu.VMEM((PPW,), jnp.int32),
        pltpu.VMEM((PCH, D), jnp.float32),
        pltpu.VMEM((PCH, D), jnp.float32),
        pltpu.SemaphoreType.DMA,
        pltpu.SemaphoreType.DMA,
    ],
)


TCR = 1024


def _tc_layer1_body(x_ref, g_ref, w_ref, o_ref):
    o_ref[...] = jnp.tanh(
        jnp.dot(x_ref[...] + g_ref[...], w_ref[...],
                preferred_element_type=jnp.float32))


def _tc_layer1(x, agg, w):
    return pl.pallas_call(
        _tc_layer1_body,
        grid=(NPAD // TCR,),
        in_specs=[
            pl.BlockSpec((TCR, D), lambda i: (i, 0)),
            pl.BlockSpec((TCR, D), lambda i: (i, 0)),
            pl.BlockSpec((D, D), lambda i: (0, 0)),
        ],
        out_specs=pl.BlockSpec((TCR, D), lambda i: (i, 0)),
        out_shape=jax.ShapeDtypeStruct((NPAD, D), jnp.float32),
    )(x, agg, w)


def _tc_layer2_body(x_ref, g_ref, w_ref, wh_ref, a_ref, b_ref):
    h2 = jnp.tanh(
        jnp.dot(x_ref[...] + g_ref[...], w_ref[...],
                preferred_element_type=jnp.float32))
    a_ref[...] = jnp.dot(h2, wh_ref[0:D, :],
                         preferred_element_type=jnp.float32)
    b_ref[...] = jnp.dot(h2, wh_ref[D:2 * D, :],
                         preferred_element_type=jnp.float32)


def _tc_layer2(x, agg, w, wh):
    return pl.pallas_call(
        _tc_layer2_body,
        grid=(NPAD // TCR,),
        in_specs=[
            pl.BlockSpec((TCR, D), lambda i: (i, 0)),
            pl.BlockSpec((TCR, D), lambda i: (i, 0)),
            pl.BlockSpec((D, D), lambda i: (0, 0)),
            pl.BlockSpec((2 * D, D), lambda i: (0, 0)),
        ],
        out_specs=[
            pl.BlockSpec((TCR, D), lambda i: (i, 0)),
            pl.BlockSpec((TCR, D), lambda i: (i, 0)),
        ],
        out_shape=[
            jax.ShapeDtypeStruct((NPAD, D), jnp.float32),
            jax.ShapeDtypeStruct((NPAD, D), jnp.float32),
        ],
    )(x, agg, w, wh)


BR = 2048


def _tc_head_body(a_ref, b_ref, lab_ref, bh_ref, wot_ref, bo_ref, loss_ref):
    h = jnp.maximum(a_ref[...] + b_ref[...] + bh_ref[...], 0.0)
    l0 = jnp.sum(h * wot_ref[0:1, :], axis=1, keepdims=True) + bo_ref[0]
    l1 = jnp.sum(h * wot_ref[1:2, :], axis=1, keepdims=True) + bo_ref[1]
    m = jnp.maximum(l0, l1)
    lse = m + jnp.log(jnp.exp(l0 - m) + jnp.exp(l1 - m))
    sel = jnp.where(lab_ref[...] == 0, l0, l1)
    part = jnp.sum(lse - sel)

    @pl.when(pl.program_id(0) == 0)
    def _():
        loss_ref[0, 0] = 0.0

    loss_ref[0, 0] += part * (1.0 / B)


def _tc_head(a, b, labels2d, bh2d, wot, bo):
    return pl.pallas_call(
        _tc_head_body,
        grid=(B // BR,),
        in_specs=[
            pl.BlockSpec((BR, D), lambda i: (i, 0)),
            pl.BlockSpec((BR, D), lambda i: (i, 0)),
            pl.BlockSpec((BR, 1), lambda i: (i, 0)),
            pl.BlockSpec((1, D), lambda i: (0, 0)),
            pl.BlockSpec((2, D), lambda i: (0, 0)),
            pl.BlockSpec(memory_space=pltpu.SMEM),
        ],
        out_specs=pl.BlockSpec(memory_space=pltpu.SMEM),
        out_shape=jax.ShapeDtypeStruct((1, 1), jnp.float32),
    )(a, b, labels2d, bh2d, wot, bo)


def kernel(pairs, labels, neighbors, edge_w, emb, W0, W1, Wh, bh, Wo, bo):
    emb_p = jnp.pad(emb, ((0, NPAD - N), (0, 0)))
    nbf = jnp.pad(neighbors, ((0, NPAD - N), (0, 0))).reshape(-1)
    ew_p = jnp.pad(edge_w, ((0, NPAD - N), (0, 0)))

    wf = _tc_prep(ew_p).reshape(-1)

    agg1 = _sc_agg(emb_p, nbf, wf)
    h1 = _tc_layer1(emb_p, agg1, W0)
    agg2 = _sc_agg(h1, nbf, wf)
    a, b = _tc_layer2(h1, agg2, W1, Wh)

    ar, br = _sc_pair(a, b, pairs[:, 0], pairs[:, 1])

    loss = _tc_head(ar, br, labels.reshape(B, 1), bh.reshape(1, D),
                    Wo.T, bo)
    return loss[0, 0]
